# baseline (speedup 1.0000x reference)
import functools

import jax
import jax.numpy as jnp
from jax import lax
from jax.experimental import pallas as pl
from jax.experimental.pallas import tpu as pltpu
from jax.experimental.pallas import tpu_sc as plsc

VOCAB = 1000000
EMB_DIM = 32
BATCH = 4096
SEQ = 50

_info = plsc.get_sparse_core_info()
NC, NS = _info.num_cores, _info.num_subcores
NW = NC * NS

FAT = 128 // EMB_DIM
N_FAT = VOCAB // FAT
SPAN = 128
N_SPANS = 999936 // SPAN
SPANS_PER_W = 246
TAIL_V0 = VOCAB - 128

B_PER_W = BATCH // NW
S_CHUNK = 10
N_S_CHUNKS = SEQ // S_CHUNK


def _iota16():
    return lax.broadcasted_iota(jnp.int32, (16,), 0)


def _transpose_span(tin, tout, n_m):
    iota = _iota16()
    for m in range(n_m):
        v = 16 * m + iota
        rows = lax.shift_right_logical(v, 2)
        cols = lax.shift_left(lax.bitwise_and(v, 3), 5)
        for d in range(EMB_DIM):
            x = tin[d, pl.ds(16 * m, 16)]
            plsc.store_scatter(tout, [rows, cols + d], x)


def _fatten_kernel(tab_t, tail_t, fat_hbm,
                   tin0, tin1, tout0, tout1, tail_in, tail_out,
                   insem, outsem):
    wid = lax.axis_index("s") * NC + lax.axis_index("c")
    sp0 = wid * SPANS_PER_W
    tin = [tin0, tin1]
    tout = [tout0, tout1]

    FROWS = SPAN // FAT

    def fire_in(t, b):
        sp = sp0 + t
        pltpu.async_copy(tab_t.at[:, pl.ds(sp * SPAN, SPAN)], tin[b], insem)

    fire_in(0, 0)

    def pair_body(tp, _):
        for b in range(2):
            t = 2 * tp + b
            sp = sp0 + t

            @pl.when(jnp.logical_and(t + 1 < SPANS_PER_W, sp + 1 < N_SPANS))
            def _():
                fire_in(t + 1, 1 - b)

            @pl.when(sp < N_SPANS)
            def _():
                @pl.when(t >= 2)
                def _():
                    pltpu.make_async_copy(
                        tout[b], fat_hbm.at[pl.ds(0, FROWS), :], outsem).wait()

                pltpu.make_async_copy(
                    tab_t.at[:, pl.ds(0, SPAN)], tin[b], insem).wait()
                _transpose_span(tin[b], tout[b], SPAN // 16)
                pltpu.async_copy(
                    tout[b], fat_hbm.at[pl.ds(sp * FROWS, FROWS), :], outsem)

        return ()

    lax.fori_loop(0, SPANS_PER_W // 2, pair_body, ())
    for _ in range(2):
        pltpu.make_async_copy(
            tout[0], fat_hbm.at[pl.ds(0, FROWS), :], outsem).wait()

    @pl.when(wid == NW - 1)
    def _():
        pltpu.sync_copy(tail_t, tail_in)
        _transpose_span(tail_in, tail_out, 8)
        pltpu.sync_copy(tail_out, fat_hbm.at[pl.ds(TAIL_V0 // FAT, 32), :])


def _lookup_kernel(idx_t, fat_hbm, out_hbm,
                   idx_v, c0, c1, fat0, fat1, fidx0, fidx1, gsem, wsem):
    wid = lax.axis_index("s") * NC + lax.axis_index("c")
    b0 = wid * B_PER_W
    cbuf = [c0, c1]
    fat_v = [fat0, fat1]
    fidx = [fidx0, fidx1]
    iota = _iota16()

    pltpu.sync_copy(idx_t.at[:, pl.ds(b0, B_PER_W)], idx_v)

    def prep_and_fire(si, b):
        for c in range(8):
            v = idx_v[si, pl.ds(16 * c, 16)]
            fidx[b][pl.ds(16 * c, 16)] = lax.shift_right_logical(v, 2)
        pltpu.async_copy(fat_hbm.at[fidx[b]], fat_v[b], gsem)

    def extract(si, si_local, b, p):
        for blk in range(8):
            ids = idx_v[si, pl.ds(16 * blk, 16)]
            colb = lax.shift_left(lax.bitwise_and(ids, 3), 5)
            rows = 16 * blk + iota
            for d in range(EMB_DIM):
                x = plsc.load_gather(fat_v[b], [rows, colb + d])
                cbuf[p][si_local, d, pl.ds(16 * blk, 16)] = x

    for ch in range(N_S_CHUNKS):
        def si_body(k, _):
            si = ch * S_CHUNK + k
            for c in range(8):
                v = idx_v[si, pl.ds(16 * c, 16)]
                fidx[0][pl.ds(16 * c, 16)] = lax.shift_right_logical(v, 2)
            pltpu.async_copy(fat_hbm.at[fidx[0]], fat_v[0], gsem).wait()
            extract(si, k, 0, 0)
            return ()

        lax.fori_loop(0, S_CHUNK, si_body, ())
        pltpu.sync_copy(
            cbuf[0],
            out_hbm.at[pl.ds(ch * S_CHUNK, S_CHUNK), :, pl.ds(b0, B_PER_W)])


@jax.jit
def _emb(word_ids, table):
    mesh = plsc.VectorSubcoreMesh(core_axis_name="c", subcore_axis_name="s")
    fatten = functools.partial(
        pl.kernel,
        mesh=mesh,
        out_type=jax.ShapeDtypeStruct((N_FAT, 128), jnp.float32),
        scratch_types=[
            pltpu.VMEM((EMB_DIM, SPAN), jnp.float32),
            pltpu.VMEM((EMB_DIM, SPAN), jnp.float32),
            pltpu.VMEM((SPAN // FAT, 128), jnp.float32),
            pltpu.VMEM((SPAN // FAT, 128), jnp.float32),
            pltpu.VMEM((EMB_DIM, 128), jnp.float32),
            pltpu.VMEM((32, 128), jnp.float32),
            pltpu.SemaphoreType.DMA,
            pltpu.SemaphoreType.DMA,
        ],
        compiler_params=pltpu.CompilerParams(needs_layout_passes=False),
    )(_fatten_kernel)
    lookup = functools.partial(
        pl.kernel,
        mesh=mesh,
        out_type=jax.ShapeDtypeStruct((SEQ, EMB_DIM, BATCH), jnp.float32),
        scratch_types=[
            pltpu.VMEM((SEQ, B_PER_W), jnp.int32),
            pltpu.VMEM((S_CHUNK, EMB_DIM, B_PER_W), jnp.float32),
            pltpu.VMEM((S_CHUNK, EMB_DIM, B_PER_W), jnp.float32),
            pltpu.VMEM((B_PER_W, 128), jnp.float32),
            pltpu.VMEM((B_PER_W, 128), jnp.float32),
            pltpu.VMEM((B_PER_W,), jnp.int32),
            pltpu.VMEM((B_PER_W,), jnp.int32),
            pltpu.SemaphoreType.DMA,
            pltpu.SemaphoreType.DMA,
        ],
        compiler_params=pltpu.CompilerParams(needs_layout_passes=False),
    )(_lookup_kernel)

    fat = fatten(table.T, table[TAIL_V0:, :].T)
    out_phys = lookup(word_ids.T, fat)
    return out_phys.transpose(2, 0, 1)


def kernel(word_ids, table):
    return _emb(word_ids, table)


# speedup vs baseline: 1.4008x; 1.4008x over previous
import functools

import jax
import jax.numpy as jnp
from jax import lax
from jax.experimental import pallas as pl
from jax.experimental.pallas import tpu as pltpu
from jax.experimental.pallas import tpu_sc as plsc

VOCAB = 1000000
EMB_DIM = 32
BATCH = 4096
SEQ = 50
TOTAL = BATCH * SEQ

_info = plsc.get_sparse_core_info()
NC, NS = _info.num_cores, _info.num_subcores
NW = NC * NS
ROWS_PER_W = BATCH // NW
ROWS_PER_CHUNK = 32
CHUNK = ROWS_PER_CHUNK * SEQ
N_CHUNKS = ROWS_PER_W // ROWS_PER_CHUNK


def _emb_kernel(idx_hbm, table_hbm, out_hbm,
                idx0, idx1, rows0, rows1, gsem, wsem):
    wid = lax.axis_index("s") * NC + lax.axis_index("c")
    row_base = wid * ROWS_PER_W
    idx_v = [idx0, idx1]
    rows_v = [rows0, rows1]

    def stage_idx(g, b):
        off = (row_base + g * ROWS_PER_CHUNK) * SEQ
        pltpu.sync_copy(idx_hbm.at[pl.ds(off, CHUNK)], idx_v[b])

    def fire_gather(g, b):
        return pltpu.async_copy(table_hbm.at[idx_v[b]], rows_v[b], gsem)

    def fire_writes(g, b):
        r0 = row_base + g * ROWS_PER_CHUNK
        return [
            pltpu.async_copy(
                rows_v[b].at[pl.ds(r * SEQ, SEQ)],
                out_hbm.at[r0 + r, :, :],
                wsem,
            )
            for r in range(ROWS_PER_CHUNK)
        ]

    stage_idx(0, 0)
    gathers = [fire_gather(0, 0)]
    writes = []
    for g in range(N_CHUNKS):
        if g + 1 < N_CHUNKS:
            b = (g + 1) % 2
            stage_idx(g + 1, b)
            if g >= 1:
                for d in writes[g - 1]:
                    d.wait()
            gathers.append(fire_gather(g + 1, b))
        gathers[g].wait()
        writes.append(fire_writes(g, g % 2))
    for d in writes[N_CHUNKS - 2]:
        d.wait()
    for d in writes[N_CHUNKS - 1]:
        d.wait()


@jax.jit
def _emb(word_ids, table):
    guarded = jnp.where(word_ids < 0, word_ids + VOCAB, word_ids)
    idx = jnp.clip(guarded, 0, VOCAB - 1).reshape(TOTAL)
    tab = jnp.minimum(table, jnp.float32(jnp.inf))
    mesh = plsc.VectorSubcoreMesh(core_axis_name="c", subcore_axis_name="s")
    k = functools.partial(
        pl.kernel,
        mesh=mesh,
        out_type=jax.ShapeDtypeStruct((BATCH, SEQ, EMB_DIM), jnp.float32),
        scratch_types=[
            pltpu.VMEM((CHUNK,), jnp.int32),
            pltpu.VMEM((CHUNK,), jnp.int32),
            pltpu.VMEM((CHUNK, EMB_DIM), jnp.float32),
            pltpu.VMEM((CHUNK, EMB_DIM), jnp.float32),
            pltpu.SemaphoreType.DMA,
            pltpu.SemaphoreType.DMA,
        ],
        compiler_params=pltpu.CompilerParams(use_tc_tiling_on_sc=False),
    )(_emb_kernel)
    return k(idx, tab)


def kernel(word_ids, table):
    return _emb(word_ids, table)
